# baseline (speedup 1.0000x reference)
import jax
import jax.numpy as jnp
from jax.experimental import pallas as pl


_COLS = 1024


def _argmin_body(x_ref, o_ref):
    k = x_ref.shape[1]
    c = x_ref.shape[2]
    nchunks = k // 8

    def step(i, carry):
        run_min, run_chunk = carry
        v = x_ref[0, pl.ds(i * 8, 8), :]
        mask = v < run_min
        run_min = jnp.where(mask, v, run_min)
        run_chunk = jnp.where(mask, i, run_chunk)
        return run_min, run_chunk

    init = (
        jnp.full((8, c), jnp.inf, jnp.float32),
        jnp.zeros((8, c), jnp.int32),
    )
    run_min, run_chunk = jax.lax.fori_loop(0, nchunks, step, init)

    mn = jnp.min(run_min, axis=0, keepdims=True)
    rows = run_chunk * 8 + jax.lax.broadcasted_iota(jnp.int32, (8, c), 0)
    big = jnp.int32(2**30)
    idx = jnp.min(jnp.where(run_min == mn, rows, big), axis=0)
    o_ref[0, 0] = idx


def kernel(x):
    b, k, n = x.shape
    grid = (b, n // _COLS)
    out = pl.pallas_call(
        _argmin_body,
        grid=grid,
        in_specs=[pl.BlockSpec((1, k, _COLS), lambda i, j: (i, 0, j))],
        out_specs=pl.BlockSpec((1, 1, _COLS), lambda i, j: (i, 0, j)),
        out_shape=jax.ShapeDtypeStruct((b, 1, n), jnp.int32),
    )(x)
    return out.reshape(b, n).astype(jnp.int64)
